# baseline (speedup 1.0000x reference)
import functools

import jax
import jax.numpy as jnp
from jax import lax
from jax.experimental import pallas as pl
from jax.experimental.pallas import tpu as pltpu
from jax.experimental.pallas import tpu_sc as plsc

_N = 50000
_K = 10
_V = 100000
_D = 128

_NC, _NS = 2, 16
_NW = _NC * _NS
_NPW = 1600
_NPAD = _NW * _NPW
_C = 32
_KP = _K + 1
_NCHUNK = _NPW // _C
_KW = 16

_NIDX = _C * _KP
_MOFF = _C * _KW
_MWORDS = 2 * _C * _KW
_STREAMS = ((0, 88), (88, 88), (176, 88), (264, 88))

_DUMMY0, _DUMMY1 = _N, _N + 2 * _C

_mesh = plsc.VectorSubcoreMesh(
    core_axis_name="c", subcore_axis_name="s", num_cores=_NC, num_subcores=_NS
)


@functools.partial(
    pl.kernel,
    out_type=jax.ShapeDtypeStruct((_NPAD, _D), jnp.float32),
    mesh=_mesh,
    scratch_types=[
        pltpu.VMEM((_NIDX,), jnp.int32),
        pltpu.VMEM((_NIDX,), jnp.int32),
        pltpu.VMEM((_MWORDS,), jnp.float32),
        pltpu.VMEM((_MWORDS,), jnp.float32),
        pltpu.VMEM((_NIDX, _D), jnp.float32),
        pltpu.VMEM((_NIDX, _D), jnp.float32),
        pltpu.VMEM((_C, _D), jnp.float32),
        pltpu.VMEM((_C, _D), jnp.float32),
        pltpu.SemaphoreType.DMA,
        pltpu.SemaphoreType.DMA,
        pltpu.SemaphoreType.DMA,
        pltpu.SemaphoreType.DMA,
        pltpu.SemaphoreType.DMA,
        pltpu.SemaphoreType.DMA,
    ],
)
def _sc_aggregate(idx_hbm, wm_hbm, table_hbm, out_hbm,
                  idx0, idx1, wm0, wm1, rows0, rows1, outv0, outv1,
                  sm0, sm1, sr0, sr1, so0, so1):
    wid = lax.axis_index("s") * _NC + lax.axis_index("c")
    mbase = wid * _NCHUNK

    def meta_descs(g, iv, wv, sem):
        return [
            pltpu.make_async_copy(
                idx_hbm.at[pl.ds((mbase + g) * _NIDX, _NIDX)], iv, sem),
            pltpu.make_async_copy(
                wm_hbm.at[pl.ds((mbase + g) * _MWORDS, _MWORDS)], wv, sem),
        ]

    def gather_descs(mv, rv, sem):
        return [
            pltpu.make_async_copy(
                table_hbm.at[mv.at[pl.ds(off, cnt)]],
                rv.at[pl.ds(off, cnt), :], sem)
            for off, cnt in _STREAMS
        ]

    def store_desc(base, ov, sem):
        return pltpu.make_async_copy(ov, out_hbm.at[pl.ds(base, _C), :], sem)

    def compute(wmref, rv, ov):
        def node_body(i, carry):
            woff = i * _KW
            wmv = wmref[pl.ds(woff, _KW)] * wmref[pl.ds(_MOFF + woff, _KW)]
            wks = [wmv[k] for k in range(_KP)]
            denom = wks[0]
            for k in range(1, _KP):
                denom = denom + wks[k]
            denom_v = jnp.full((16,), denom, jnp.float32)
            inv = 1.0 / jnp.maximum(denom_v, 1e-6)
            rbase = i * _KP
            acc = [None] * (_D // 16)
            for k in range(_KP):
                wk = wks[k]
                for d in range(_D // 16):
                    seg = rv[rbase + k, pl.ds(d * 16, 16)]
                    acc[d] = wk * seg if k == 0 else acc[d] + wk * seg
            for d in range(_D // 16):
                ov[i, pl.ds(d * 16, 16)] = acc[d] * inv
            return carry

        lax.fori_loop(0, _C, node_body, 0)

    for d in meta_descs(0, idx0, wm0, sm0):
        d.start()
    for d in meta_descs(1, idx1, wm1, sm1):
        d.start()
    store_desc(_DUMMY0, outv0, so0).start()
    store_desc(_DUMMY1, outv1, so1).start()
    for d in meta_descs(0, idx0, wm0, sm0):
        d.wait()
    for d in gather_descs(idx0, rows0, sr0):
        d.start()

    def chunk_pair(i, carry):
        cA = 2 * i
        cB = 2 * i + 1
        for d in meta_descs(lax.rem(cA + 1, _NCHUNK), idx1, wm1, sm1):
            d.wait()
        for d in gather_descs(idx1, rows1, sr1):
            d.start()
        for d in gather_descs(idx0, rows0, sr0):
            d.wait()
        baseA_prev = jnp.where(cA >= 2, wid * _NPW + (cA - 2) * _C, _DUMMY0)
        store_desc(baseA_prev, outv0, so0).wait()
        compute(wm0, rows0, outv0)
        store_desc(wid * _NPW + cA * _C, outv0, so0).start()
        for d in meta_descs(lax.rem(cA + 2, _NCHUNK), idx0, wm0, sm0):
            d.start()
        for d in meta_descs(lax.rem(cB + 1, _NCHUNK), idx0, wm0, sm0):
            d.wait()
        for d in gather_descs(idx0, rows0, sr0):
            d.start()
        for d in gather_descs(idx1, rows1, sr1):
            d.wait()
        baseB_prev = jnp.where(cB >= 2, wid * _NPW + (cB - 2) * _C, _DUMMY1)
        store_desc(baseB_prev, outv1, so1).wait()
        compute(wm1, rows1, outv1)
        store_desc(wid * _NPW + cB * _C, outv1, so1).start()
        for d in meta_descs(lax.rem(cB + 2, _NCHUNK), idx1, wm1, sm1):
            d.start()
        return carry

    lax.fori_loop(0, _NCHUNK // 2, chunk_pair, 0)

    for d in meta_descs(1, idx1, wm1, sm1):
        d.wait()
    for d in gather_descs(idx0, rows0, sr0):
        d.wait()
    store_desc(wid * _NPW + (_NCHUNK - 2) * _C, outv0, so0).wait()
    store_desc(wid * _NPW + (_NCHUNK - 1) * _C, outv1, so1).wait()


def kernel(embed_weight, neighbor_weight, neighbor_mask, nodes_pocket, neighbor_idx):
    idx_all = jnp.concatenate(
        [neighbor_idx.astype(jnp.int32),
         nodes_pocket.astype(jnp.int32)[:, None]], axis=1)
    idx_all = jnp.pad(idx_all, ((0, _NPAD - _N), (0, 0)))
    ones = jnp.ones((_N, 1), jnp.float32)
    w_all = jnp.pad(jnp.concatenate([neighbor_weight, ones], axis=1),
                    ((0, _NPAD - _N), (0, _KW - _KP)))
    m_all = jnp.pad(jnp.concatenate([neighbor_mask, ones], axis=1),
                    ((0, _NPAD - _N), (0, _KW - _KP)))
    idx_c = idx_all.reshape(_NW, _NCHUNK, _C * _KP).reshape(-1)
    wm_c = jnp.concatenate(
        [w_all.reshape(_NW, _NCHUNK, _C * _KW),
         m_all.reshape(_NW, _NCHUNK, _C * _KW)], axis=2).reshape(-1)
    out = _sc_aggregate(idx_c, wm_c, embed_weight)
    return out[:_N]


# speedup vs baseline: 2.6305x; 2.6305x over previous
import functools

import jax
import jax.numpy as jnp
from jax import lax
from jax.experimental import pallas as pl
from jax.experimental.pallas import tpu as pltpu
from jax.experimental.pallas import tpu_sc as plsc

_N = 50000
_K = 10
_V = 100000
_D = 128

_NC, _NS = 2, 16
_C = 32
_TOTC = 1600
_CPS = _TOTC // _NS
_NC0, _NC1 = 82, 18
_NI = _C * _K
_STREAMS = ((0, 128), (128, 128), (256, 64))

_mesh = plsc.VectorSubcoreMesh(
    core_axis_name="c", subcore_axis_name="s", num_cores=_NC, num_subcores=_NS
)


@functools.partial(
    pl.kernel,
    out_type=jax.ShapeDtypeStruct((_N, _D), jnp.float32),
    mesh=_mesh,
    scratch_types=[
        pltpu.VMEM((_NI,), jnp.int32),
        pltpu.VMEM((_NI,), jnp.int32),
        pltpu.VMEM((_C,), jnp.int32),
        pltpu.VMEM((_C,), jnp.int32),
        pltpu.VMEM((_NI + 16,), jnp.float32),
        pltpu.VMEM((_NI + 16,), jnp.float32),
        pltpu.VMEM((_NI + 16,), jnp.float32),
        pltpu.VMEM((_NI + 16,), jnp.float32),
        pltpu.VMEM((_NI, _D), jnp.float32),
        pltpu.VMEM((_NI, _D), jnp.float32),
        pltpu.VMEM((_C, _D), jnp.float32),
        pltpu.VMEM((_C, _D), jnp.float32),
        pltpu.VMEM((_C, _D), jnp.float32),
        pltpu.VMEM((_C, _D), jnp.float32),
        pltpu.SemaphoreType.DMA,
        pltpu.SemaphoreType.DMA,
        pltpu.SemaphoreType.DMA,
        pltpu.SemaphoreType.DMA,
        pltpu.SemaphoreType.DMA,
        pltpu.SemaphoreType.DMA,
    ],
)
def _sc_aggregate(nidx_hbm, pocket_hbm, w_hbm, m_hbm, table_hbm, out_hbm,
                  nidx0, nidx1, self0, self1, w0, w1, m0, m1,
                  nrows0, nrows1, srows0, srows1, outv0, outv1,
                  sm0, sm1, sr0, sr1, so0, so1):
    core = lax.axis_index("c")
    sub = lax.axis_index("s")
    chunk0 = sub * _CPS + jnp.where(core == 0, 0, _NC0)
    n_my = jnp.where(core == 0, _NC0, _NC1)

    def node_base(g_local):
        return jnp.minimum((chunk0 + g_local) * _C, _N - _C)

    def meta_descs(g_local, iv, sv, wv, mv, sem):
        nb = node_base(g_local)
        return [
            pltpu.make_async_copy(nidx_hbm.at[pl.ds(nb * _K, _NI)], iv, sem),
            pltpu.make_async_copy(pocket_hbm.at[pl.ds(nb, _C)], sv, sem),
            pltpu.make_async_copy(w_hbm.at[pl.ds(nb * _K, _NI)],
                                  wv.at[pl.ds(0, _NI)], sem),
            pltpu.make_async_copy(m_hbm.at[pl.ds(nb * _K, _NI)],
                                  mv.at[pl.ds(0, _NI)], sem),
        ]

    def gather_descs(iv, sv, nr, srws, sem):
        descs = [
            pltpu.make_async_copy(
                table_hbm.at[iv.at[pl.ds(off, cnt)]],
                nr.at[pl.ds(off, cnt), :], sem)
            for off, cnt in _STREAMS
        ]
        descs.append(pltpu.make_async_copy(table_hbm.at[sv], srws, sem))
        return descs

    def store_desc(g_local, ov, sem):
        return pltpu.make_async_copy(
            ov, out_hbm.at[pl.ds(node_base(g_local), _C), :], sem)

    lanes = lax.iota(jnp.int32, 16)
    valid = lanes < _K

    def compute(wv, mv, nr, srws, ov):
        def node_body(i, carry):
            wvec = wv[pl.ds(i * _K, 16)]
            mvec = mv[pl.ds(i * _K, 16)]
            wmv = jnp.where(valid, wvec * mvec, 0.0)
            wks = [wmv[k] for k in range(_K)]
            denom = 1.0 + wks[0]
            for k in range(1, _K):
                denom = denom + wks[k]
            inv = 1.0 / jnp.maximum(jnp.full((16,), denom, jnp.float32), 1e-6)
            rbase = i * _K
            acc = [None] * (_D // 16)
            for d in range(_D // 16):
                acc[d] = srows_seg = srws[i, pl.ds(d * 16, 16)]
            for k in range(_K):
                wk = wks[k]
                for d in range(_D // 16):
                    acc[d] = acc[d] + wk * nr[rbase + k, pl.ds(d * 16, 16)]
            for d in range(_D // 16):
                ov[i, pl.ds(d * 16, 16)] = acc[d] * inv
            return carry

        lax.fori_loop(0, _C, node_body, 0)

    for d in meta_descs(0, nidx0, self0, w0, m0, sm0):
        d.start()
    for d in meta_descs(1, nidx1, self1, w1, m1, sm1):
        d.start()
    for d in meta_descs(0, nidx0, self0, w0, m0, sm0):
        d.wait()
    for d in gather_descs(nidx0, self0, nrows0, srows0, sr0):
        d.start()

    def chunk_pair(i, carry):
        cA = 2 * i
        cB = 2 * i + 1
        for d in meta_descs(lax.rem(cA + 1, n_my), nidx1, self1, w1, m1, sm1):
            d.wait()
        for d in gather_descs(nidx1, self1, nrows1, srows1, sr1):
            d.start()
        for d in gather_descs(nidx0, self0, nrows0, srows0, sr0):
            d.wait()

        @pl.when(cA >= 2)
        def _():
            store_desc(cA - 2, outv0, so0).wait()

        compute(w0, m0, nrows0, srows0, outv0)
        store_desc(cA, outv0, so0).start()
        for d in meta_descs(lax.rem(cA + 2, n_my), nidx0, self0, w0, m0, sm0):
            d.start()
        for d in meta_descs(lax.rem(cB + 1, n_my), nidx0, self0, w0, m0, sm0):
            d.wait()
        for d in gather_descs(nidx0, self0, nrows0, srows0, sr0):
            d.start()
        for d in gather_descs(nidx1, self1, nrows1, srows1, sr1):
            d.wait()

        @pl.when(cB >= 2)
        def _():
            store_desc(cB - 2, outv1, so1).wait()

        compute(w1, m1, nrows1, srows1, outv1)
        store_desc(cB, outv1, so1).start()
        for d in meta_descs(lax.rem(cB + 2, n_my), nidx1, self1, w1, m1, sm1):
            d.start()
        return carry

    lax.fori_loop(0, n_my // 2, chunk_pair, 0)

    for d in meta_descs(1, nidx1, self1, w1, m1, sm1):
        d.wait()
    for d in gather_descs(nidx0, self0, nrows0, srows0, sr0):
        d.wait()
    store_desc(n_my - 2, outv0, so0).wait()
    store_desc(n_my - 1, outv1, so1).wait()


def kernel(embed_weight, neighbor_weight, neighbor_mask, nodes_pocket, neighbor_idx):
    nidx = neighbor_idx.astype(jnp.int32).reshape(-1)
    pocket = nodes_pocket.astype(jnp.int32)
    return _sc_aggregate(nidx, pocket, neighbor_weight.reshape(-1),
                         neighbor_mask.reshape(-1), embed_weight)


# speedup vs baseline: 3.9667x; 1.5080x over previous
import functools

import jax
import jax.numpy as jnp
from jax import lax
from jax.experimental import pallas as pl
from jax.experimental.pallas import tpu as pltpu
from jax.experimental.pallas import tpu_sc as plsc

_N = 50000
_K = 10
_V = 100000
_D = 128

_NC, _NS = 2, 16
_C = 32
_TOTC = 1600
_CPS = _TOTC // _NS
_NC0, _NC1 = 50, 50
_NI = _C * _K
_STREAMS = ((0, 128), (128, 128), (256, 64))

_mesh = plsc.VectorSubcoreMesh(
    core_axis_name="c", subcore_axis_name="s", num_cores=_NC, num_subcores=_NS
)


@functools.partial(
    pl.kernel,
    out_type=jax.ShapeDtypeStruct((_N, _D), jnp.float32),
    mesh=_mesh,
    scratch_types=[
        pltpu.VMEM((_NI,), jnp.int32),
        pltpu.VMEM((_NI,), jnp.int32),
        pltpu.VMEM((_C,), jnp.int32),
        pltpu.VMEM((_C,), jnp.int32),
        pltpu.VMEM((_NI + 16,), jnp.float32),
        pltpu.VMEM((_NI + 16,), jnp.float32),
        pltpu.VMEM((_NI, _D), jnp.float32),
        pltpu.VMEM((_NI, _D), jnp.float32),
        pltpu.VMEM((_C, _D), jnp.float32),
        pltpu.VMEM((_C, _D), jnp.float32),
        pltpu.VMEM((_C, _D), jnp.float32),
        pltpu.VMEM((_C, _D), jnp.float32),
        pltpu.SemaphoreType.DMA,
        pltpu.SemaphoreType.DMA,
        pltpu.SemaphoreType.DMA,
        pltpu.SemaphoreType.DMA,
        pltpu.SemaphoreType.DMA,
        pltpu.SemaphoreType.DMA,
    ],
)
def _sc_aggregate(nidx_hbm, pocket_hbm, wm_hbm, table_hbm, out_hbm,
                  nidx0, nidx1, self0, self1, wm0, wm1,
                  nrows0, nrows1, srows0, srows1, outv0, outv1,
                  sm0, sm1, sr0, sr1, so0, so1):
    core = lax.axis_index("c")
    sub = lax.axis_index("s")
    chunk0 = sub * _CPS + jnp.where(core == 0, 0, _NC0)
    n_my = jnp.where(core == 0, _NC0, _NC1)

    def node_base(g_local):
        return jnp.minimum((chunk0 + g_local) * _C, _N - _C)

    def meta_descs(g_local, iv, sv, wv, sem):
        nb = node_base(g_local)
        return [
            pltpu.make_async_copy(nidx_hbm.at[pl.ds(nb * _K, _NI)], iv, sem),
            pltpu.make_async_copy(pocket_hbm.at[pl.ds(nb, _C)], sv, sem),
            pltpu.make_async_copy(wm_hbm.at[pl.ds(nb * _K, _NI)],
                                  wv.at[pl.ds(0, _NI)], sem),
        ]

    def gather_descs(iv, sv, nr, srws, sem):
        descs = [
            pltpu.make_async_copy(
                table_hbm.at[iv.at[pl.ds(off, cnt)]],
                nr.at[pl.ds(off, cnt), :], sem)
            for off, cnt in _STREAMS
        ]
        descs.append(pltpu.make_async_copy(table_hbm.at[sv], srws, sem))
        return descs

    def store_desc(g_local, ov, sem):
        return pltpu.make_async_copy(
            ov, out_hbm.at[pl.ds(node_base(g_local), _C), :], sem)

    lanes = lax.iota(jnp.int32, 16)
    valid = lanes < _K

    def compute(wv, nr, srws, ov):
        def node_body(i, carry):
            wmv = jnp.where(valid, wv[pl.ds(i * _K, 16)], 0.0)
            wks = [wmv[k] for k in range(_K)]
            denom = 1.0 + wks[0]
            for k in range(1, _K):
                denom = denom + wks[k]
            inv = 1.0 / jnp.maximum(jnp.full((16,), denom, jnp.float32), 1e-6)
            rbase = i * _K
            acc = [None] * (_D // 16)
            for d in range(_D // 16):
                acc[d] = srws[i, pl.ds(d * 16, 16)]
            for k in range(_K):
                wk = wks[k]
                for d in range(_D // 16):
                    acc[d] = acc[d] + wk * nr[rbase + k, pl.ds(d * 16, 16)]
            for d in range(_D // 16):
                ov[i, pl.ds(d * 16, 16)] = acc[d] * inv
            return carry

        lax.fori_loop(0, _C, node_body, 0)

    for d in meta_descs(0, nidx0, self0, wm0, sm0):
        d.start()
    for d in meta_descs(1, nidx1, self1, wm1, sm1):
        d.start()
    for d in meta_descs(0, nidx0, self0, wm0, sm0):
        d.wait()
    for d in gather_descs(nidx0, self0, nrows0, srows0, sr0):
        d.start()

    def chunk_pair(i, carry):
        cA = 2 * i
        cB = 2 * i + 1
        for d in meta_descs(lax.rem(cA + 1, n_my), nidx1, self1, wm1, sm1):
            d.wait()
        for d in gather_descs(nidx1, self1, nrows1, srows1, sr1):
            d.start()
        for d in gather_descs(nidx0, self0, nrows0, srows0, sr0):
            d.wait()

        @pl.when(cA >= 2)
        def _():
            store_desc(cA - 2, outv0, so0).wait()

        compute(wm0, nrows0, srows0, outv0)
        store_desc(cA, outv0, so0).start()
        for d in meta_descs(lax.rem(cA + 2, n_my), nidx0, self0, wm0, sm0):
            d.start()
        for d in meta_descs(lax.rem(cB + 1, n_my), nidx0, self0, wm0, sm0):
            d.wait()
        for d in gather_descs(nidx0, self0, nrows0, srows0, sr0):
            d.start()
        for d in gather_descs(nidx1, self1, nrows1, srows1, sr1):
            d.wait()

        @pl.when(cB >= 2)
        def _():
            store_desc(cB - 2, outv1, so1).wait()

        compute(wm1, nrows1, srows1, outv1)
        store_desc(cB, outv1, so1).start()
        for d in meta_descs(lax.rem(cB + 2, n_my), nidx1, self1, wm1, sm1):
            d.start()
        return carry

    lax.fori_loop(0, n_my // 2, chunk_pair, 0)

    for d in meta_descs(1, nidx1, self1, wm1, sm1):
        d.wait()
    for d in gather_descs(nidx0, self0, nrows0, srows0, sr0):
        d.wait()
    store_desc(n_my - 2, outv0, so0).wait()
    store_desc(n_my - 1, outv1, so1).wait()


def kernel(embed_weight, neighbor_weight, neighbor_mask, nodes_pocket, neighbor_idx):
    nidx = neighbor_idx.astype(jnp.int32).reshape(-1)
    wm = (neighbor_weight * neighbor_mask).reshape(-1)
    return _sc_aggregate(nidx, nodes_pocket.astype(jnp.int32), wm,
                         embed_weight)
